# baseline (speedup 1.0000x reference)
import functools

import jax
import jax.numpy as jnp
from jax import lax
from jax.experimental import pallas as pl
from jax.experimental.pallas import tpu as pltpu
from jax.experimental.pallas import tpu_sc as plsc

N = 10000
E = 320000
D = 128

NC = 2
NS = 16
NW = NC * NS
K = 128

N_PAD = 10112
RPT = N_PAD // NS
NBUF = 2
NBLK = 80
CCH = 16
NCH = NBLK // CCH
E_PAD = NW * NBLK * K
EPW = NBLK * K

_mesh = plsc.VectorSubcoreMesh(core_axis_name="c", subcore_axis_name="s")


@functools.partial(
    pl.kernel,
    out_type=jax.ShapeDtypeStruct((NC, 2, N_PAD, 8), jnp.float32),
    mesh=_mesh,
    scratch_types=[
        pltpu.VMEM((NBLK, K), jnp.int32),
        pltpu.VMEM((NBLK, K), jnp.int32),
        pltpu.VMEM((K, 8), jnp.float32),
        pltpu.VMEM_SHARED((N_PAD, 8), jnp.float32),
        pltpu.VMEM_SHARED((N_PAD, 8), jnp.float32),
    ],
)
def _deg_kernel(src_hbm, dst_hbm, ones_hbm, zeros_hbm, out_hbm,
                src_v, dst_v, ones_v, acc_out, acc_in):
    cid = lax.axis_index("c")
    sid = lax.axis_index("s")
    wid = cid * NS + sid

    rows = pl.ds(sid * RPT, RPT)
    pltpu.sync_copy(zeros_hbm, acc_out.at[rows])
    pltpu.sync_copy(zeros_hbm, acc_in.at[rows])
    pltpu.sync_copy(ones_hbm, ones_v)
    pltpu.sync_copy(src_hbm.at[wid], src_v)
    pltpu.sync_copy(dst_hbm.at[wid], dst_v)
    plsc.subcore_barrier()

    def body(j, carry):
        pltpu.sync_copy(ones_v, acc_out.at[src_v.at[j]], add=True)
        pltpu.sync_copy(ones_v, acc_in.at[dst_v.at[j]], add=True)
        return carry

    lax.fori_loop(0, NBLK, body, 0)
    plsc.subcore_barrier()

    pltpu.sync_copy(acc_out.at[rows], out_hbm.at[cid, 0, rows])
    pltpu.sync_copy(acc_in.at[rows], out_hbm.at[cid, 1, rows])


@functools.partial(
    pl.kernel,
    out_type=jax.ShapeDtypeStruct((NC, N_PAD, D), jnp.float32),
    mesh=_mesh,
    scratch_types=[
        pltpu.VMEM((NBLK, K), jnp.int32),
        pltpu.VMEM((2, CCH, K), jnp.int32),
        pltpu.VMEM((K, D), jnp.float32),
        pltpu.VMEM((K, D), jnp.float32),
        pltpu.SemaphoreType.DMA,
        pltpu.SemaphoreType.DMA,
        pltpu.SemaphoreType.DMA,
        pltpu.SemaphoreType.DMA,
        pltpu.VMEM_SHARED((N_PAD, D), jnp.float32),
    ],
)
def _agg_kernel(y_hbm, src_hbm, dst_hbm, zeros_hbm, out_hbm,
                src_v, dst_v, b0, b1, sg0, sg1, si0, si1, acc):
    cid = lax.axis_index("c")
    sid = lax.axis_index("s")
    wid = cid * NS + sid
    bufs = [b0, b1]
    semg = [sg0, sg1]
    semi = [si0, si1]

    def g_issue(j, b):
        pltpu.async_copy(y_hbm.at[src_v.at[j]], bufs[b], semg[b])

    def g_wait(j, b):
        pltpu.make_async_copy(y_hbm.at[src_v.at[j]], bufs[b], semg[b]).wait()

    def s_sync(j, b, c, r):
        pltpu.sync_copy(bufs[b], acc.at[dst_v.at[c & 1, r]], add=True)

    def i_issue(c):
        pltpu.async_copy(dst_hbm.at[wid, pl.ds(c * CCH, CCH)],
                         dst_v.at[c & 1], semi[c & 1])

    def i_wait(c):
        pltpu.make_async_copy(dst_hbm.at[wid, pl.ds(c * CCH, CCH)],
                              dst_v.at[c & 1], semi[c & 1]).wait()

    rows = pl.ds(sid * RPT, RPT)
    pltpu.sync_copy(zeros_hbm, acc.at[rows])
    pltpu.sync_copy(src_hbm.at[wid], src_v)
    i_issue(0)
    plsc.subcore_barrier()
    i_wait(0)

    g_issue(0, 0)
    for c in range(NCH):
        if c + 1 < NCH:
            i_issue(c + 1)
        j0 = c * CCH
        npairs = CCH // 2 if c + 1 < NCH else CCH // 2 - 1

        def pair(g, carry, j0=j0, c=c):
            j = j0 + 2 * g
            g_wait(j, 0)
            g_issue(j + 1, 1)
            s_sync(j, 0, c, 2 * g)
            g_wait(j + 1, 1)
            g_issue(j + 2, 0)
            s_sync(j + 1, 1, c, 2 * g + 1)
            return carry

        lax.fori_loop(0, npairs, pair, 0)
        if c + 1 < NCH:
            i_wait(c + 1)
        else:
            g_wait(NBLK - 2, 0)
            g_issue(NBLK - 1, 1)
            s_sync(NBLK - 2, 0, c, CCH - 2)
            g_wait(NBLK - 1, 1)
            s_sync(NBLK - 1, 1, c, CCH - 1)

    plsc.subcore_barrier()
    pltpu.sync_copy(acc.at[rows], out_hbm.at[cid, rows])


def _scale_vecs(degp):
    dego = degp[0, 0, :, 0:1] + degp[1, 0, :, 0:1]
    degi = degp[0, 1, :, 0:1] + degp[1, 1, :, 0:1]
    a = lax.rsqrt(jnp.maximum(dego, 1.0))
    c = lax.rsqrt(jnp.maximum(degi, 1.0))
    return a, c


def _tc1_body(degp_ref, x_ref, w_ref, y_ref):
    a, _ = _scale_vecs(degp_ref[...])
    xw = jnp.dot(x_ref[...], w_ref[...], preferred_element_type=jnp.float32)
    y_ref[...] = xw * a


def _tc2_body(degp_ref, p_ref, b_ref, w_ref, y_ref):
    a, c = _scale_vecs(degp_ref[...])
    h = jnp.maximum((p_ref[0] + p_ref[1]) * c + b_ref[...], 0.0)
    y_ref[...] = jnp.dot(h, w_ref[...], preferred_element_type=jnp.float32) * a


def _tc3_body(degp_ref, p_ref, b_ref, o_ref):
    _, c = _scale_vecs(degp_ref[...])
    o_ref[...] = (p_ref[0] + p_ref[1]) * c + b_ref[...]


_f32 = jnp.float32
_tc1 = pl.pallas_call(_tc1_body, out_shape=jax.ShapeDtypeStruct((N_PAD, D), _f32))
_tc2 = pl.pallas_call(_tc2_body, out_shape=jax.ShapeDtypeStruct((N_PAD, D), _f32))
_tc3 = pl.pallas_call(_tc3_body, out_shape=jax.ShapeDtypeStruct((N_PAD, D), _f32))


@jax.jit
def kernel(G, x, W1, b1, W2, b2):
    src = G[0]
    dst = G[1]
    pad = jnp.full((E_PAD - E,), N, dtype=jnp.int32)
    src3 = jnp.concatenate([src, pad]).reshape(NW, NBLK, K)
    dst3 = jnp.concatenate([dst, pad]).reshape(NW, NBLK, K)
    x_pad = jnp.zeros((N_PAD, D), _f32).at[:N].set(x)

    ones8 = jnp.zeros((K, 8), _f32).at[:, 0].set(1.0)
    zeros8 = jnp.zeros((RPT, 8), _f32)
    zrows = jnp.zeros((RPT, D), _f32)

    degp = _deg_kernel(src3, dst3, ones8, zeros8)
    y1 = _tc1(degp, x_pad, W1)
    p1 = _agg_kernel(y1, src3, dst3, zrows)
    y2 = _tc2(degp, p1, b1.reshape(1, D), W2)
    p2 = _agg_kernel(y2, src3, dst3, zrows)
    out = _tc3(degp, p2, b2.reshape(1, D))
    return out[:N]
